# baseline (speedup 1.0000x reference)
import dataclasses
import functools

import jax
import jax.numpy as jnp
from jax import lax
from jax.experimental import pallas as pl
from jax.experimental.pallas import tpu as pltpu
from jax.experimental.pallas import tpu_sc as plsc

H = 512
W = 512
NPIX = float(H * W)
NTILES = 32
ROWS_PER_TILE = H // NTILES
NBINS = 100
LOG2 = 0.6931471805599453
L1 = 0.3132616875182228
SIG1 = 0.7310585786300049


def _prep_kernel(p_ref, t_ref, cls_ref, ppo_ref, g_ref, s_ref):
    best = p_ref[0]
    cls = jnp.zeros((H, W), jnp.int32)
    for c in range(1, 5):
        pc = p_ref[c]
        m = pc > best
        best = jnp.where(m, pc, best)
        cls = jnp.where(m, c, cls)
    ppo = jnp.where(cls > 0, best, 0.0)
    gterm = jnp.maximum(ppo, 0.0) + jnp.log1p(jnp.exp(-jnp.abs(ppo)))

    cls_ref[...] = cls
    ppo_ref[...] = ppo
    g_ref[...] = gterm

    tgt = t_ref[...]
    colzero = jnp.max(cls, axis=0, keepdims=True) == 0
    t0 = tgt == 0
    s_ref[0, 0] = jnp.sum(colzero.astype(jnp.float32))
    s_ref[0, 1] = jnp.sum(jnp.where(colzero & t0, 1.0, 0.0))
    s_ref[0, 2] = jnp.sum(t0.astype(jnp.float32))


def _ccl_kernel(cls_ref, ncc_ref):
    cls = cls_ref[...]
    row = lax.broadcasted_iota(jnp.int32, (H, W), 0)
    col = lax.broadcasted_iota(jnp.int32, (H, W), 1)
    idx = row * W + col + 1
    lab0 = jnp.where(cls > 0, idx, 0)

    adj1 = (jnp.where(row == 0, -1, jnp.roll(cls, 1, 0)) == cls)
    adj2 = (jnp.where(row == H - 1, -1, jnp.roll(cls, -1, 0)) == cls)
    adj3 = (jnp.where(col == 0, -1, jnp.roll(cls, 1, 1)) == cls)
    adj4 = (jnp.where(col == W - 1, -1, jnp.roll(cls, -1, 1)) == cls)

    def step(lab):
        m = lab
        m = jnp.maximum(m, jnp.where(adj1, jnp.roll(lab, 1, 0), 0))
        m = jnp.maximum(m, jnp.where(adj2, jnp.roll(lab, -1, 0), 0))
        m = jnp.maximum(m, jnp.where(adj3, jnp.roll(lab, 1, 1), 0))
        m = jnp.maximum(m, jnp.where(adj4, jnp.roll(lab, -1, 1), 0))
        return m

    def body(carry):
        lab, _ = carry
        new = step(step(lab))
        return new, jnp.any(new != lab)

    lab, _ = lax.while_loop(lambda c: c[1], body, (lab0, jnp.bool_(True)))

    roots = lab == idx
    for v in range(1, 5):
        ncc_ref[0, v - 1] = jnp.sum(((cls == v) & roots).astype(jnp.int32))


def _sc_stats(cls, tgt, ppo, gterm):
    mesh = plsc.VectorSubcoreMesh(core_axis_name="c", subcore_axis_name="s")
    cp = pltpu.CompilerParams()
    if "needs_layout_passes" in pltpu.CompilerParams.__dataclass_fields__:
        cp = dataclasses.replace(cp, needs_layout_passes=False)

    @functools.partial(
        pl.kernel,
        mesh=mesh,
        compiler_params=cp,
        out_type=jax.ShapeDtypeStruct((NTILES, 16 * NBINS), jnp.float32),
        scratch_types=[
            pltpu.VMEM((ROWS_PER_TILE, W), jnp.int32),
            pltpu.VMEM((ROWS_PER_TILE, W), jnp.int32),
            pltpu.VMEM((ROWS_PER_TILE, W), jnp.float32),
            pltpu.VMEM((ROWS_PER_TILE, W), jnp.float32),
            pltpu.VMEM((16 * NBINS,), jnp.float32),
        ],
    )
    def stats_kernel(cls_hbm, tgt_hbm, ppo_hbm, g_hbm, out_hbm,
                     cls_v, tgt_v, ppo_v, g_v, acc):
        wid = lax.axis_index("s") * 2 + lax.axis_index("c")
        r0 = wid * ROWS_PER_TILE
        pltpu.sync_copy(cls_hbm.at[pl.ds(r0, ROWS_PER_TILE)], cls_v)
        pltpu.sync_copy(tgt_hbm.at[pl.ds(r0, ROWS_PER_TILE)], tgt_v)
        pltpu.sync_copy(ppo_hbm.at[pl.ds(r0, ROWS_PER_TILE)], ppo_v)
        pltpu.sync_copy(g_hbm.at[pl.ds(r0, ROWS_PER_TILE)], g_v)

        zeros16 = jnp.zeros((16,), jnp.float32)
        ones16 = jnp.ones((16,), jnp.float32)
        laneoff = lax.broadcasted_iota(jnp.int32, (16,), 0) * NBINS

        @pl.loop(0, 16 * NBINS, step=16)
        def _(i):
            acc[pl.ds(i, 16)] = zeros16

        @pl.loop(0, ROWS_PER_TILE)
        def _(r):
            @pl.loop(0, W, step=16)
            def _(j):
                c16 = cls_v[r, pl.ds(j, 16)]
                t16 = tgt_v[r, pl.ds(j, 16)]
                v16 = ppo_v[r, pl.ds(j, 16)]
                g16 = g_v[r, pl.ds(j, 16)]
                key = c16 * 5 + t16
                idx = laneoff + key
                sg = ones16 / (ones16 + jnp.exp(-v16))
                plsc.addupdate_scatter(acc, [idx], ones16)
                plsc.addupdate_scatter(acc, [idx + 25], v16)
                plsc.addupdate_scatter(acc, [idx + 50], sg)
                plsc.addupdate_scatter(acc, [idx + 75], g16)

        pltpu.sync_copy(acc, out_hbm.at[wid])

    return stats_kernel(cls, tgt, ppo, gterm)


def _final_kernel(b_ref, ncc_ref, s_ref, o_ref):
    b2 = jnp.sum(b_ref[...], axis=0, keepdims=True)

    def bv(k):
        return b2[0, k]

    Z, q, n0t = s_ref[0, 0], s_ref[0, 1], s_ref[0, 2]
    bce0 = ((NPIX - H * Z) * LOG2 + H * Z * (1.0 + L1) - q) / NPIX
    inter0 = SIG1 * q + 0.5 * (n0t - q)
    sumP0 = SIG1 * H * Z + 0.5 * (NPIX - H * Z)
    dice0 = 1.0 - (2.0 * inter0 + 1.0) / (sumP0 + n0t + 1.0)
    res = bce0 + dice0

    n_ct = [[bv(c * 5 + t).astype(jnp.int32) for t in range(5)]
            for c in range(5)]
    B_ct = [[bv(25 + c * 5 + t) for t in range(5)] for c in range(5)]
    S_ct = [[bv(50 + c * 5 + t) for t in range(5)] for c in range(5)]
    A_c = [bv(75 + c * 5 + 0) + bv(75 + c * 5 + 1) + bv(75 + c * 5 + 2)
           + bv(75 + c * 5 + 3) + bv(75 + c * 5 + 4) for c in range(5)]
    n_c = [sum(n_ct[c][1:], n_ct[c][0]) for c in range(5)]
    S_c = [sum(S_ct[c][1:], S_ct[c][0]) for c in range(5)]

    last_i = jnp.int32(1)
    a = [jnp.float32(0.0)] * 5
    for v in range(1, 5):
        present = n_c[v] > 0
        n_nz = ncc_ref[0, v - 1]
        n_uniq = n_nz + (n_c[v] < H * W).astype(jnp.int32)
        s_v = (n_nz * last_i).astype(jnp.float32)
        a = [(a[c] + (1.0 if c == v else 0.0)) + s_v for c in range(5)]
        last_i = last_i + jnp.where(present, n_uniq, 0)

    for t in range(1, 5):
        n = n_ct[0][t] + n_ct[1][t] + n_ct[2][t] + n_ct[3][t] + n_ct[4][t]
        k = (n - 1) // 2
        med = jnp.float32(jnp.inf)
        for c in range(5):
            cum = jnp.int32(0)
            for c2 in range(5):
                cum = cum + jnp.where(a[c2] <= a[c], n_ct[c2][t], 0)
            med = jnp.minimum(med, jnp.where(cum >= k + 1, a[c],
                                             jnp.float32(jnp.inf)))
        nM = jnp.float32(0.0)
        sumA = jnp.float32(0.0)
        sumB_Mt = jnp.float32(0.0)
        sumS_M = jnp.float32(0.0)
        sumS_Mt = jnp.float32(0.0)
        n_Mt = jnp.float32(0.0)
        sumB_t = jnp.float32(0.0)
        for c in range(5):
            match = a[c] == med
            nM = nM + jnp.where(match, n_c[c].astype(jnp.float32), 0.0)
            sumA = sumA + jnp.where(match, A_c[c], 0.0)
            sumB_Mt = sumB_Mt + jnp.where(match, B_ct[c][t], 0.0)
            sumS_M = sumS_M + jnp.where(match, S_c[c], 0.0)
            sumS_Mt = sumS_Mt + jnp.where(match, S_ct[c][t], 0.0)
            n_Mt = n_Mt + jnp.where(match, n_ct[c][t].astype(jnp.float32),
                                    0.0)
            sumB_t = sumB_t + B_ct[c][t]
        nf = n.astype(jnp.float32)
        bce = (sumA + (NPIX - nM) * LOG2 - sumB_Mt) / NPIX
        inter = sumS_Mt + 0.5 * (nf - n_Mt)
        sumP = sumS_M + 0.5 * (NPIX - nM)
        dice = 1.0 - (2.0 * inter + 1.0) / (sumP + nf + 1.0)
        extra = (sumB_t - sumB_Mt) / nf
        contrib = bce + dice + extra
        res = res + jnp.where(n > 0, contrib, 0.0)

    n_t_total = jnp.int32(0)
    for t in range(5):
        cnt_t = (n_ct[0][t] + n_ct[1][t] + n_ct[2][t] + n_ct[3][t]
                 + n_ct[4][t])
        n_t_total = n_t_total + (cnt_t > 0).astype(jnp.int32)

    o_ref[0, 0] = res / (n_t_total * 2 + 1).astype(jnp.float32)


@jax.jit
def _run(pred_out, target_mask):
    p = pred_out.reshape(5, H, W)
    tgt = target_mask.reshape(H, W)

    cls, ppo, gterm, s = pl.pallas_call(
        _prep_kernel,
        out_shape=(
            jax.ShapeDtypeStruct((H, W), jnp.int32),
            jax.ShapeDtypeStruct((H, W), jnp.float32),
            jax.ShapeDtypeStruct((H, W), jnp.float32),
            jax.ShapeDtypeStruct((1, 4), jnp.float32),
        ),
        in_specs=[
            pl.BlockSpec(memory_space=pltpu.VMEM),
            pl.BlockSpec(memory_space=pltpu.VMEM),
        ],
        out_specs=(
            pl.BlockSpec(memory_space=pltpu.VMEM),
            pl.BlockSpec(memory_space=pltpu.VMEM),
            pl.BlockSpec(memory_space=pltpu.VMEM),
            pl.BlockSpec(memory_space=pltpu.SMEM),
        ),
    )(p, tgt)

    bins = _sc_stats(cls, tgt, ppo, gterm)

    ncc = pl.pallas_call(
        _ccl_kernel,
        out_shape=jax.ShapeDtypeStruct((1, 4), jnp.int32),
        in_specs=[pl.BlockSpec(memory_space=pltpu.VMEM)],
        out_specs=pl.BlockSpec(memory_space=pltpu.SMEM),
    )(cls)

    out = pl.pallas_call(
        _final_kernel,
        out_shape=jax.ShapeDtypeStruct((1, 1), jnp.float32),
        in_specs=[
            pl.BlockSpec(memory_space=pltpu.VMEM),
            pl.BlockSpec(memory_space=pltpu.SMEM),
            pl.BlockSpec(memory_space=pltpu.SMEM),
        ],
        out_specs=pl.BlockSpec(memory_space=pltpu.SMEM),
    )(bins.reshape(NTILES * 16, NBINS), ncc, s)
    return out[0, 0]


def kernel(pred_out, target_mask):
    return _run(pred_out, target_mask)


# speedup vs baseline: 1.0458x; 1.0458x over previous
import dataclasses
import functools

import jax
import jax.numpy as jnp
from jax import lax
from jax.experimental import pallas as pl
from jax.experimental.pallas import tpu as pltpu
from jax.experimental.pallas import tpu_sc as plsc

H = 512
W = 512
NPIX = float(H * W)
NTILES = 32
ROWS_PER_TILE = H // NTILES
NBINS = 100
LOG2 = 0.6931471805599453
L1 = 0.3132616875182228
SIG1 = 0.7310585786300049


def _prep_kernel(p_ref, t_ref, cls_ref, ppo_ref, g_ref, s_ref):
    best = p_ref[0]
    cls = jnp.zeros((H, W), jnp.int32)
    for c in range(1, 5):
        pc = p_ref[c]
        m = pc > best
        best = jnp.where(m, pc, best)
        cls = jnp.where(m, c, cls)
    ppo = jnp.where(cls > 0, best, 0.0)
    gterm = jnp.maximum(ppo, 0.0) + jnp.log1p(jnp.exp(-jnp.abs(ppo)))

    cls_ref[...] = cls
    ppo_ref[...] = ppo
    g_ref[...] = gterm

    tgt = t_ref[...]
    colzero = jnp.max(cls, axis=0, keepdims=True) == 0
    t0 = tgt == 0
    s_ref[0, 0] = jnp.sum(colzero.astype(jnp.float32))
    s_ref[0, 1] = jnp.sum(jnp.where(colzero & t0, 1.0, 0.0))
    s_ref[0, 2] = jnp.sum(t0.astype(jnp.float32))


def _ccl_kernel(cls_ref, ncc_ref):
    cls = cls_ref[...]
    row = lax.broadcasted_iota(jnp.int32, (H, W), 0)
    col = lax.broadcasted_iota(jnp.int32, (H, W), 1)
    idx = row * W + col + 1
    lab0 = jnp.where(cls > 0, idx, 0)

    adj1 = (jnp.where(row == 0, -1, jnp.roll(cls, 1, 0)) == cls)
    adj2 = (jnp.where(row == H - 1, -1, jnp.roll(cls, -1, 0)) == cls)
    adj3 = (jnp.where(col == 0, -1, jnp.roll(cls, 1, 1)) == cls)
    adj4 = (jnp.where(col == W - 1, -1, jnp.roll(cls, -1, 1)) == cls)

    def step(lab):
        m = lab
        m = jnp.maximum(m, jnp.where(adj1, jnp.roll(lab, 1, 0), 0))
        m = jnp.maximum(m, jnp.where(adj2, jnp.roll(lab, -1, 0), 0))
        m = jnp.maximum(m, jnp.where(adj3, jnp.roll(lab, 1, 1), 0))
        m = jnp.maximum(m, jnp.where(adj4, jnp.roll(lab, -1, 1), 0))
        return m

    def body(carry):
        lab, _ = carry
        new = step(step(lab))
        return new, jnp.any(new != lab)

    lab, _ = lax.while_loop(lambda c: c[1], body, (lab0, jnp.bool_(True)))

    roots = lab == idx
    for v in range(1, 5):
        ncc_ref[0, v - 1] = jnp.sum(((cls == v) & roots).astype(jnp.int32))


def _sc_stats(cls, tgt, ppo, gterm):
    mesh = plsc.VectorSubcoreMesh(core_axis_name="c", subcore_axis_name="s")
    cp = pltpu.CompilerParams()
    if "needs_layout_passes" in pltpu.CompilerParams.__dataclass_fields__:
        cp = dataclasses.replace(cp, needs_layout_passes=False)

    @functools.partial(
        pl.kernel,
        mesh=mesh,
        compiler_params=cp,
        out_type=jax.ShapeDtypeStruct((NTILES, 16, NBINS), jnp.float32),
        scratch_types=[
            pltpu.VMEM((ROWS_PER_TILE, W), jnp.int32),
            pltpu.VMEM((ROWS_PER_TILE, W), jnp.int32),
            pltpu.VMEM((ROWS_PER_TILE, W), jnp.float32),
            pltpu.VMEM((ROWS_PER_TILE, W), jnp.float32),
            pltpu.VMEM((16, NBINS), jnp.float32),
        ],
    )
    def stats_kernel(cls_hbm, tgt_hbm, ppo_hbm, g_hbm, out_hbm,
                     cls_v, tgt_v, ppo_v, g_v, acc):
        wid = lax.axis_index("s") * 2 + lax.axis_index("c")
        r0 = wid * ROWS_PER_TILE
        pltpu.sync_copy(cls_hbm.at[pl.ds(r0, ROWS_PER_TILE)], cls_v)
        pltpu.sync_copy(tgt_hbm.at[pl.ds(r0, ROWS_PER_TILE)], tgt_v)
        pltpu.sync_copy(ppo_hbm.at[pl.ds(r0, ROWS_PER_TILE)], ppo_v)
        pltpu.sync_copy(g_hbm.at[pl.ds(r0, ROWS_PER_TILE)], g_v)

        zeros16 = jnp.zeros((16,), jnp.float32)
        ones16 = jnp.ones((16,), jnp.float32)
        lane = lax.broadcasted_iota(jnp.int32, (16,), 0)

        @pl.loop(0, 16)
        def _(i):
            @pl.loop(0, NBINS, step=16)
            def _(j):
                acc[i, pl.ds(j, 16)] = zeros16

        @pl.loop(0, ROWS_PER_TILE)
        def _(r):
            @pl.loop(0, W, step=16)
            def _(j):
                c16 = cls_v[r, pl.ds(j, 16)]
                t16 = tgt_v[r, pl.ds(j, 16)]
                v16 = ppo_v[r, pl.ds(j, 16)]
                g16 = g_v[r, pl.ds(j, 16)]
                key = c16 * 5 + t16
                sg = ones16 / (ones16 + jnp.exp(-v16))
                plsc.addupdate_scatter(acc, [lane, key], ones16)
                plsc.addupdate_scatter(acc, [lane, key + 25], v16)
                plsc.addupdate_scatter(acc, [lane, key + 50], sg)
                plsc.addupdate_scatter(acc, [lane, key + 75], g16)

        pltpu.sync_copy(acc, out_hbm.at[wid])

    return stats_kernel(cls, tgt, ppo, gterm)


def _final_kernel(b_ref, ncc_ref, s_ref, o_ref):
    b2 = jnp.sum(b_ref[...], axis=0, keepdims=True)

    def bv(k):
        return b2[0, k]

    Z, q, n0t = s_ref[0, 0], s_ref[0, 1], s_ref[0, 2]
    bce0 = ((NPIX - H * Z) * LOG2 + H * Z * (1.0 + L1) - q) / NPIX
    inter0 = SIG1 * q + 0.5 * (n0t - q)
    sumP0 = SIG1 * H * Z + 0.5 * (NPIX - H * Z)
    dice0 = 1.0 - (2.0 * inter0 + 1.0) / (sumP0 + n0t + 1.0)
    res = bce0 + dice0

    n_ct = [[bv(c * 5 + t).astype(jnp.int32) for t in range(5)]
            for c in range(5)]
    B_ct = [[bv(25 + c * 5 + t) for t in range(5)] for c in range(5)]
    S_ct = [[bv(50 + c * 5 + t) for t in range(5)] for c in range(5)]
    A_c = [bv(75 + c * 5 + 0) + bv(75 + c * 5 + 1) + bv(75 + c * 5 + 2)
           + bv(75 + c * 5 + 3) + bv(75 + c * 5 + 4) for c in range(5)]
    n_c = [sum(n_ct[c][1:], n_ct[c][0]) for c in range(5)]
    S_c = [sum(S_ct[c][1:], S_ct[c][0]) for c in range(5)]

    last_i = jnp.int32(1)
    a = [jnp.float32(0.0)] * 5
    for v in range(1, 5):
        present = n_c[v] > 0
        n_nz = ncc_ref[0, v - 1]
        n_uniq = n_nz + (n_c[v] < H * W).astype(jnp.int32)
        s_v = (n_nz * last_i).astype(jnp.float32)
        a = [(a[c] + (1.0 if c == v else 0.0)) + s_v for c in range(5)]
        last_i = last_i + jnp.where(present, n_uniq, 0)

    for t in range(1, 5):
        n = n_ct[0][t] + n_ct[1][t] + n_ct[2][t] + n_ct[3][t] + n_ct[4][t]
        k = (n - 1) // 2
        med = jnp.float32(jnp.inf)
        for c in range(5):
            cum = jnp.int32(0)
            for c2 in range(5):
                cum = cum + jnp.where(a[c2] <= a[c], n_ct[c2][t], 0)
            med = jnp.minimum(med, jnp.where(cum >= k + 1, a[c],
                                             jnp.float32(jnp.inf)))
        nM = jnp.float32(0.0)
        sumA = jnp.float32(0.0)
        sumB_Mt = jnp.float32(0.0)
        sumS_M = jnp.float32(0.0)
        sumS_Mt = jnp.float32(0.0)
        n_Mt = jnp.float32(0.0)
        sumB_t = jnp.float32(0.0)
        for c in range(5):
            match = a[c] == med
            nM = nM + jnp.where(match, n_c[c].astype(jnp.float32), 0.0)
            sumA = sumA + jnp.where(match, A_c[c], 0.0)
            sumB_Mt = sumB_Mt + jnp.where(match, B_ct[c][t], 0.0)
            sumS_M = sumS_M + jnp.where(match, S_c[c], 0.0)
            sumS_Mt = sumS_Mt + jnp.where(match, S_ct[c][t], 0.0)
            n_Mt = n_Mt + jnp.where(match, n_ct[c][t].astype(jnp.float32),
                                    0.0)
            sumB_t = sumB_t + B_ct[c][t]
        nf = n.astype(jnp.float32)
        bce = (sumA + (NPIX - nM) * LOG2 - sumB_Mt) / NPIX
        inter = sumS_Mt + 0.5 * (nf - n_Mt)
        sumP = sumS_M + 0.5 * (NPIX - nM)
        dice = 1.0 - (2.0 * inter + 1.0) / (sumP + nf + 1.0)
        extra = (sumB_t - sumB_Mt) / nf
        contrib = bce + dice + extra
        res = res + jnp.where(n > 0, contrib, 0.0)

    n_t_total = jnp.int32(0)
    for t in range(5):
        cnt_t = (n_ct[0][t] + n_ct[1][t] + n_ct[2][t] + n_ct[3][t]
                 + n_ct[4][t])
        n_t_total = n_t_total + (cnt_t > 0).astype(jnp.int32)

    o_ref[0, 0] = res / (n_t_total * 2 + 1).astype(jnp.float32)


@jax.jit
def _run(pred_out, target_mask):
    p = pred_out.reshape(5, H, W)
    tgt = target_mask.reshape(H, W)

    cls, ppo, gterm, s = pl.pallas_call(
        _prep_kernel,
        out_shape=(
            jax.ShapeDtypeStruct((H, W), jnp.int32),
            jax.ShapeDtypeStruct((H, W), jnp.float32),
            jax.ShapeDtypeStruct((H, W), jnp.float32),
            jax.ShapeDtypeStruct((1, 4), jnp.float32),
        ),
        in_specs=[
            pl.BlockSpec(memory_space=pltpu.VMEM),
            pl.BlockSpec(memory_space=pltpu.VMEM),
        ],
        out_specs=(
            pl.BlockSpec(memory_space=pltpu.VMEM),
            pl.BlockSpec(memory_space=pltpu.VMEM),
            pl.BlockSpec(memory_space=pltpu.VMEM),
            pl.BlockSpec(memory_space=pltpu.SMEM),
        ),
    )(p, tgt)

    bins = _sc_stats(cls, tgt, ppo, gterm)

    ncc = pl.pallas_call(
        _ccl_kernel,
        out_shape=jax.ShapeDtypeStruct((1, 4), jnp.int32),
        in_specs=[pl.BlockSpec(memory_space=pltpu.VMEM)],
        out_specs=pl.BlockSpec(memory_space=pltpu.SMEM),
    )(cls)

    out = pl.pallas_call(
        _final_kernel,
        out_shape=jax.ShapeDtypeStruct((1, 1), jnp.float32),
        in_specs=[
            pl.BlockSpec(memory_space=pltpu.VMEM),
            pl.BlockSpec(memory_space=pltpu.SMEM),
            pl.BlockSpec(memory_space=pltpu.SMEM),
        ],
        out_specs=pl.BlockSpec(memory_space=pltpu.SMEM),
    )(bins.reshape(NTILES * 16, NBINS), ncc, s)
    return out[0, 0]


def kernel(pred_out, target_mask):
    return _run(pred_out, target_mask)


# speedup vs baseline: 1.3404x; 1.2818x over previous
import functools

import jax
import jax.numpy as jnp
from jax.experimental import pallas as pl
from jax.experimental.pallas import tpu as pltpu

H = 512
W = 512
NPIX = float(H * W)


def _shift(x, axis, shift):
    return jnp.roll(x, shift, axis=axis)


def _loss_kernel(p_ref, t_ref, o_ref):
    import numpy as np

    p0 = p_ref[0]
    best = p0
    cls = jnp.zeros((H, W), jnp.int32)
    for c in range(1, 5):
        pc = p_ref[c]
        m = pc > best
        best = jnp.where(m, pc, best)
        cls = jnp.where(m, c, cls)

    ppo = jnp.where(cls > 0, best, 0.0)

    tgt = t_ref[...]

    sig = jax.nn.sigmoid(ppo)
    gterm = jnp.maximum(ppo, 0.0) + jnp.log1p(jnp.exp(-jnp.abs(ppo)))

    LOG2 = 0.6931471805599453
    L1 = 0.3132616875182228
    SIG1 = 0.7310585786300049

    colmax = jnp.max(cls, axis=0, keepdims=True)
    colzero = colmax == 0
    t0 = tgt == 0
    Z = jnp.sum(colzero.astype(jnp.float32))
    q = jnp.sum(jnp.where(colzero & t0, 1.0, 0.0))
    n0t = jnp.sum(t0.astype(jnp.float32))
    bce0 = ((NPIX - H * Z) * LOG2 + H * Z * (1.0 + L1) - q) / NPIX
    inter0 = SIG1 * q + 0.5 * (n0t - q)
    sumP0 = SIG1 * H * Z + 0.5 * (NPIX - H * Z)
    dice0 = 1.0 - (2.0 * inter0 + 1.0) / (sumP0 + n0t + 1.0)
    res = bce0 + dice0

    n_c, A_c, S_c = [], [], []
    n_ct, B_ct, S_ct = [], [], []
    for c in range(5):
        mc = cls == c
        n_c.append(jnp.sum(mc.astype(jnp.int32)))
        A_c.append(jnp.sum(jnp.where(mc, gterm, 0.0)))
        S_c.append(jnp.sum(jnp.where(mc, sig, 0.0)))
        nr, Br, Sr = [], [], []
        for t in range(5):
            mct = mc & (tgt == t)
            nr.append(jnp.sum(mct.astype(jnp.int32)))
            Br.append(jnp.sum(jnp.where(mct, ppo, 0.0)))
            Sr.append(jnp.sum(jnp.where(mct, sig, 0.0)))
        n_ct.append(nr)
        B_ct.append(Br)
        S_ct.append(Sr)

    row = jax.lax.broadcasted_iota(jnp.int32, (H, W), 0)
    col = jax.lax.broadcasted_iota(jnp.int32, (H, W), 1)
    idx = row * W + col + 1
    lab0 = jnp.where(cls > 0, idx, 0)

    adj1 = (jnp.where(row == 0, -1, _shift(cls, 0, 1)) == cls)
    adj2 = (jnp.where(row == H - 1, -1, _shift(cls, 0, -1)) == cls)
    adj3 = (jnp.where(col == 0, -1, _shift(cls, 1, 1)) == cls)
    adj4 = (jnp.where(col == W - 1, -1, _shift(cls, 1, -1)) == cls)

    def step(lab):
        m = lab
        m = jnp.maximum(m, jnp.where(adj1, _shift(lab, 0, 1), 0))
        m = jnp.maximum(m, jnp.where(adj2, _shift(lab, 0, -1), 0))
        m = jnp.maximum(m, jnp.where(adj3, _shift(lab, 1, 1), 0))
        m = jnp.maximum(m, jnp.where(adj4, _shift(lab, 1, -1), 0))
        return m

    lab0 = step(step(step(step(lab0))))
    lab0 = step(step(step(step(lab0))))
    lab0 = step(step(step(step(lab0))))

    def body(carry):
        lab, _ = carry
        new = step(step(lab))
        return new, jnp.any(new != lab)

    def cond(carry):
        return carry[1]

    lab, _ = jax.lax.while_loop(cond, body, (lab0, jnp.bool_(True)))

    ncc = []
    roots = lab == idx
    for v in range(1, 5):
        ncc.append(jnp.sum(((cls == v) & roots).astype(jnp.int32)))

    last_i = jnp.int32(1)
    a = [jnp.float32(0.0)] * 5
    for v in range(1, 5):
        present = n_c[v] > 0
        n_nz = ncc[v - 1]
        n_uniq = n_nz + (n_c[v] < H * W).astype(jnp.int32)
        s_v = (n_nz * last_i).astype(jnp.float32)
        a = [(a[c] + (1.0 if c == v else 0.0)) + s_v for c in range(5)]
        last_i = last_i + jnp.where(present, n_uniq, 0)

    for t in range(1, 5):
        n = n_ct[0][t] + n_ct[1][t] + n_ct[2][t] + n_ct[3][t] + n_ct[4][t]
        k = (n - 1) // 2
        med = jnp.float32(jnp.inf)
        for c in range(5):
            cum = jnp.int32(0)
            for c2 in range(5):
                cum = cum + jnp.where(a[c2] <= a[c], n_ct[c2][t], 0)
            med = jnp.minimum(med, jnp.where(cum >= k + 1, a[c],
                                             jnp.float32(jnp.inf)))
        nM = jnp.float32(0.0)
        sumA = jnp.float32(0.0)
        sumB_Mt = jnp.float32(0.0)
        sumS_M = jnp.float32(0.0)
        sumS_Mt = jnp.float32(0.0)
        n_Mt = jnp.float32(0.0)
        sumB_t = jnp.float32(0.0)
        for c in range(5):
            match = a[c] == med
            nM = nM + jnp.where(match, n_c[c].astype(jnp.float32), 0.0)
            sumA = sumA + jnp.where(match, A_c[c], 0.0)
            sumB_Mt = sumB_Mt + jnp.where(match, B_ct[c][t], 0.0)
            sumS_M = sumS_M + jnp.where(match, S_c[c], 0.0)
            sumS_Mt = sumS_Mt + jnp.where(match, S_ct[c][t], 0.0)
            n_Mt = n_Mt + jnp.where(match, n_ct[c][t].astype(jnp.float32), 0.0)
            sumB_t = sumB_t + B_ct[c][t]
        nf = n.astype(jnp.float32)
        bce = (sumA + (NPIX - nM) * LOG2 - sumB_Mt) / NPIX
        inter = sumS_Mt + 0.5 * (nf - n_Mt)
        sumP = sumS_M + 0.5 * (NPIX - nM)
        dice = 1.0 - (2.0 * inter + 1.0) / (sumP + nf + 1.0)
        extra = (sumB_t - sumB_Mt) / nf
        contrib = bce + dice + extra
        res = res + jnp.where(n > 0, contrib, 0.0)

    n_t_total = jnp.int32(0)
    for t in range(5):
        cnt_t = (n_ct[0][t] + n_ct[1][t] + n_ct[2][t] + n_ct[3][t]
                 + n_ct[4][t])
        n_t_total = n_t_total + (cnt_t > 0).astype(jnp.int32)

    o_ref[0, 0] = res / (n_t_total * 2 + 1).astype(jnp.float32)


@functools.partial(jax.jit, static_argnames=("interpret",))
def _run(pred_out, target_mask, interpret=False):
    p = pred_out.reshape(5, H, W)
    tgt = target_mask.reshape(H, W)
    out = pl.pallas_call(
        _loss_kernel,
        out_shape=jax.ShapeDtypeStruct((1, 1), jnp.float32),
        in_specs=[
            pl.BlockSpec(memory_space=pltpu.VMEM),
            pl.BlockSpec(memory_space=pltpu.VMEM),
        ],
        out_specs=pl.BlockSpec(memory_space=pltpu.SMEM),
        interpret=interpret,
    )(p, tgt)
    return out[0, 0]


def kernel(pred_out, target_mask):
    return _run(pred_out, target_mask)


# speedup vs baseline: 1.3506x; 1.0076x over previous
import functools

import jax
import jax.numpy as jnp
from jax.experimental import pallas as pl
from jax.experimental.pallas import tpu as pltpu

H = 512
W = 512
NPIX = float(H * W)


def _shift(x, axis, shift):
    return jnp.roll(x, shift, axis=axis)


def _loss_kernel(p_ref, t_ref, o_ref):
    import numpy as np

    p0 = p_ref[0]
    best = p0
    cls = jnp.zeros((H, W), jnp.int32)
    for c in range(1, 5):
        pc = p_ref[c]
        m = pc > best
        best = jnp.where(m, pc, best)
        cls = jnp.where(m, c, cls)

    ppo = jnp.where(cls > 0, best, 0.0)

    tgt = t_ref[...]

    sig = jax.nn.sigmoid(ppo)
    gterm = jnp.maximum(ppo, 0.0) + jnp.log1p(jnp.exp(-jnp.abs(ppo)))

    LOG2 = 0.6931471805599453
    L1 = 0.3132616875182228
    SIG1 = 0.7310585786300049

    colmax = jnp.max(cls, axis=0, keepdims=True)
    colzero = colmax == 0
    t0 = tgt == 0
    Z = jnp.sum(colzero.astype(jnp.float32))
    q = jnp.sum(jnp.where(colzero & t0, 1.0, 0.0))
    n0t = jnp.sum(t0.astype(jnp.float32))
    bce0 = ((NPIX - H * Z) * LOG2 + H * Z * (1.0 + L1) - q) / NPIX
    inter0 = SIG1 * q + 0.5 * (n0t - q)
    sumP0 = SIG1 * H * Z + 0.5 * (NPIX - H * Z)
    dice0 = 1.0 - (2.0 * inter0 + 1.0) / (sumP0 + n0t + 1.0)
    res = bce0 + dice0

    n_c, A_c, S_c = [], [], []
    n_ct, B_ct, S_ct = [], [], []
    for c in range(5):
        mc = cls == c
        n_c.append(jnp.sum(mc.astype(jnp.int32)))
        A_c.append(jnp.sum(jnp.where(mc, gterm, 0.0)))
        S_c.append(jnp.sum(jnp.where(mc, sig, 0.0)))
        nr, Br, Sr = [], [], []
        for t in range(5):
            mct = mc & (tgt == t)
            nr.append(jnp.sum(mct.astype(jnp.int32)))
            Br.append(jnp.sum(jnp.where(mct, ppo, 0.0)))
            Sr.append(jnp.sum(jnp.where(mct, sig, 0.0)))
        n_ct.append(nr)
        B_ct.append(Br)
        S_ct.append(Sr)

    row = jax.lax.broadcasted_iota(jnp.int32, (H, W), 0)
    col = jax.lax.broadcasted_iota(jnp.int32, (H, W), 1)
    idx = row * W + col + 1
    lab0 = jnp.where(cls > 0, idx, 0)

    adj1 = (jnp.where(row == 0, -1, _shift(cls, 0, 1)) == cls)
    adj2 = (jnp.where(row == H - 1, -1, _shift(cls, 0, -1)) == cls)
    adj3 = (jnp.where(col == 0, -1, _shift(cls, 1, 1)) == cls)
    adj4 = (jnp.where(col == W - 1, -1, _shift(cls, 1, -1)) == cls)

    def step(lab):
        m = lab
        m = jnp.maximum(m, jnp.where(adj1, _shift(lab, 0, 1), 0))
        m = jnp.maximum(m, jnp.where(adj2, _shift(lab, 0, -1), 0))
        m = jnp.maximum(m, jnp.where(adj3, _shift(lab, 1, 1), 0))
        m = jnp.maximum(m, jnp.where(adj4, _shift(lab, 1, -1), 0))
        return m

    lab0 = step(step(step(step(lab0))))
    lab0 = step(step(step(step(lab0))))
    lab0 = step(step(step(step(lab0))))
    lab0 = step(step(lab0))

    def body(carry):
        lab, _ = carry
        new = step(step(lab))
        return new, jnp.any(new != lab)

    def cond(carry):
        return carry[1]

    lab, _ = jax.lax.while_loop(cond, body, (lab0, jnp.bool_(True)))

    ncc = []
    roots = lab == idx
    for v in range(1, 5):
        ncc.append(jnp.sum(((cls == v) & roots).astype(jnp.int32)))

    last_i = jnp.int32(1)
    a = [jnp.float32(0.0)] * 5
    for v in range(1, 5):
        present = n_c[v] > 0
        n_nz = ncc[v - 1]
        n_uniq = n_nz + (n_c[v] < H * W).astype(jnp.int32)
        s_v = (n_nz * last_i).astype(jnp.float32)
        a = [(a[c] + (1.0 if c == v else 0.0)) + s_v for c in range(5)]
        last_i = last_i + jnp.where(present, n_uniq, 0)

    for t in range(1, 5):
        n = n_ct[0][t] + n_ct[1][t] + n_ct[2][t] + n_ct[3][t] + n_ct[4][t]
        k = (n - 1) // 2
        med = jnp.float32(jnp.inf)
        for c in range(5):
            cum = jnp.int32(0)
            for c2 in range(5):
                cum = cum + jnp.where(a[c2] <= a[c], n_ct[c2][t], 0)
            med = jnp.minimum(med, jnp.where(cum >= k + 1, a[c],
                                             jnp.float32(jnp.inf)))
        nM = jnp.float32(0.0)
        sumA = jnp.float32(0.0)
        sumB_Mt = jnp.float32(0.0)
        sumS_M = jnp.float32(0.0)
        sumS_Mt = jnp.float32(0.0)
        n_Mt = jnp.float32(0.0)
        sumB_t = jnp.float32(0.0)
        for c in range(5):
            match = a[c] == med
            nM = nM + jnp.where(match, n_c[c].astype(jnp.float32), 0.0)
            sumA = sumA + jnp.where(match, A_c[c], 0.0)
            sumB_Mt = sumB_Mt + jnp.where(match, B_ct[c][t], 0.0)
            sumS_M = sumS_M + jnp.where(match, S_c[c], 0.0)
            sumS_Mt = sumS_Mt + jnp.where(match, S_ct[c][t], 0.0)
            n_Mt = n_Mt + jnp.where(match, n_ct[c][t].astype(jnp.float32), 0.0)
            sumB_t = sumB_t + B_ct[c][t]
        nf = n.astype(jnp.float32)
        bce = (sumA + (NPIX - nM) * LOG2 - sumB_Mt) / NPIX
        inter = sumS_Mt + 0.5 * (nf - n_Mt)
        sumP = sumS_M + 0.5 * (NPIX - nM)
        dice = 1.0 - (2.0 * inter + 1.0) / (sumP + nf + 1.0)
        extra = (sumB_t - sumB_Mt) / nf
        contrib = bce + dice + extra
        res = res + jnp.where(n > 0, contrib, 0.0)

    n_t_total = jnp.int32(0)
    for t in range(5):
        cnt_t = (n_ct[0][t] + n_ct[1][t] + n_ct[2][t] + n_ct[3][t]
                 + n_ct[4][t])
        n_t_total = n_t_total + (cnt_t > 0).astype(jnp.int32)

    o_ref[0, 0] = res / (n_t_total * 2 + 1).astype(jnp.float32)


@functools.partial(jax.jit, static_argnames=("interpret",))
def _run(pred_out, target_mask, interpret=False):
    p = pred_out.reshape(5, H, W)
    tgt = target_mask.reshape(H, W)
    out = pl.pallas_call(
        _loss_kernel,
        out_shape=jax.ShapeDtypeStruct((1, 1), jnp.float32),
        in_specs=[
            pl.BlockSpec(memory_space=pltpu.VMEM),
            pl.BlockSpec(memory_space=pltpu.VMEM),
        ],
        out_specs=pl.BlockSpec(memory_space=pltpu.SMEM),
        interpret=interpret,
    )(p, tgt)
    return out[0, 0]


def kernel(pred_out, target_mask):
    return _run(pred_out, target_mask)


# speedup vs baseline: 1.3955x; 1.0333x over previous
import functools

import jax
import jax.numpy as jnp
from jax.experimental import pallas as pl
from jax.experimental.pallas import tpu as pltpu

H = 512
W = 512
NPIX = float(H * W)


def _shift(x, axis, shift):
    return jnp.roll(x, shift, axis=axis)


def _loss_kernel(p_ref, t_ref, o_ref):
    import numpy as np

    p0 = p_ref[0]
    best = p0
    cls = jnp.zeros((H, W), jnp.int32)
    for c in range(1, 5):
        pc = p_ref[c]
        m = pc > best
        best = jnp.where(m, pc, best)
        cls = jnp.where(m, c, cls)

    ppo = jnp.where(cls > 0, best, 0.0)

    tgt = t_ref[...]

    sig = jax.nn.sigmoid(ppo)
    gterm = jnp.maximum(ppo, 0.0) + jnp.log1p(jnp.exp(-jnp.abs(ppo)))

    LOG2 = 0.6931471805599453
    L1 = 0.3132616875182228
    SIG1 = 0.7310585786300049

    colmax = jnp.max(cls, axis=0, keepdims=True)
    colzero = colmax == 0
    t0 = tgt == 0
    Z = jnp.sum(colzero.astype(jnp.float32))
    q = jnp.sum(jnp.where(colzero & t0, 1.0, 0.0))
    n0t = jnp.sum(t0.astype(jnp.float32))
    bce0 = ((NPIX - H * Z) * LOG2 + H * Z * (1.0 + L1) - q) / NPIX
    inter0 = SIG1 * q + 0.5 * (n0t - q)
    sumP0 = SIG1 * H * Z + 0.5 * (NPIX - H * Z)
    dice0 = 1.0 - (2.0 * inter0 + 1.0) / (sumP0 + n0t + 1.0)
    res = bce0 + dice0

    n_c, A_c, S_c = [], [], []
    n_ct, B_ct, S_ct = [], [], []
    for c in range(5):
        mc = cls == c
        A_c.append(jnp.sum(jnp.where(mc, gterm, 0.0)))
        nr, Br, Sr = [], [], []
        for t in range(5):
            mct = mc & (tgt == t)
            nr.append(jnp.sum(mct.astype(jnp.int32)))
            Br.append(jnp.sum(jnp.where(mct, ppo, 0.0)))
            Sr.append(jnp.sum(jnp.where(mct, sig, 0.0)))
        n_ct.append(nr)
        B_ct.append(Br)
        S_ct.append(Sr)
        n_c.append(nr[0] + nr[1] + nr[2] + nr[3] + nr[4])
        S_c.append(Sr[0] + Sr[1] + Sr[2] + Sr[3] + Sr[4])

    row = jax.lax.broadcasted_iota(jnp.int32, (H, W), 0)
    col = jax.lax.broadcasted_iota(jnp.int32, (H, W), 1)
    idx = row * W + col + 1
    lab0 = jnp.where(cls > 0, idx, 0)

    adj1 = (jnp.where(row == 0, -1, _shift(cls, 0, 1)) == cls)
    adj2 = (jnp.where(row == H - 1, -1, _shift(cls, 0, -1)) == cls)
    adj3 = (jnp.where(col == 0, -1, _shift(cls, 1, 1)) == cls)
    adj4 = (jnp.where(col == W - 1, -1, _shift(cls, 1, -1)) == cls)

    def step(lab):
        m = lab
        m = jnp.maximum(m, jnp.where(adj1, _shift(lab, 0, 1), 0))
        m = jnp.maximum(m, jnp.where(adj2, _shift(lab, 0, -1), 0))
        m = jnp.maximum(m, jnp.where(adj3, _shift(lab, 1, 1), 0))
        m = jnp.maximum(m, jnp.where(adj4, _shift(lab, 1, -1), 0))
        return m

    lab0 = step(step(step(step(lab0))))
    lab0 = step(step(step(step(lab0))))
    lab0 = step(step(step(step(lab0))))
    lab0 = step(step(lab0))

    def body(carry):
        lab, _ = carry
        new = step(lab)
        return new, jnp.any(new != lab)

    def cond(carry):
        return carry[1]

    lab, _ = jax.lax.while_loop(cond, body, (lab0, jnp.bool_(True)))

    ncc = []
    roots = lab == idx
    for v in range(1, 5):
        ncc.append(jnp.sum(((cls == v) & roots).astype(jnp.int32)))

    last_i = jnp.int32(1)
    a = [jnp.float32(0.0)] * 5
    for v in range(1, 5):
        present = n_c[v] > 0
        n_nz = ncc[v - 1]
        n_uniq = n_nz + (n_c[v] < H * W).astype(jnp.int32)
        s_v = (n_nz * last_i).astype(jnp.float32)
        a = [(a[c] + (1.0 if c == v else 0.0)) + s_v for c in range(5)]
        last_i = last_i + jnp.where(present, n_uniq, 0)

    for t in range(1, 5):
        n = n_ct[0][t] + n_ct[1][t] + n_ct[2][t] + n_ct[3][t] + n_ct[4][t]
        k = (n - 1) // 2
        med = jnp.float32(jnp.inf)
        for c in range(5):
            cum = jnp.int32(0)
            for c2 in range(5):
                cum = cum + jnp.where(a[c2] <= a[c], n_ct[c2][t], 0)
            med = jnp.minimum(med, jnp.where(cum >= k + 1, a[c],
                                             jnp.float32(jnp.inf)))
        nM = jnp.float32(0.0)
        sumA = jnp.float32(0.0)
        sumB_Mt = jnp.float32(0.0)
        sumS_M = jnp.float32(0.0)
        sumS_Mt = jnp.float32(0.0)
        n_Mt = jnp.float32(0.0)
        sumB_t = jnp.float32(0.0)
        for c in range(5):
            match = a[c] == med
            nM = nM + jnp.where(match, n_c[c].astype(jnp.float32), 0.0)
            sumA = sumA + jnp.where(match, A_c[c], 0.0)
            sumB_Mt = sumB_Mt + jnp.where(match, B_ct[c][t], 0.0)
            sumS_M = sumS_M + jnp.where(match, S_c[c], 0.0)
            sumS_Mt = sumS_Mt + jnp.where(match, S_ct[c][t], 0.0)
            n_Mt = n_Mt + jnp.where(match, n_ct[c][t].astype(jnp.float32), 0.0)
            sumB_t = sumB_t + B_ct[c][t]
        nf = n.astype(jnp.float32)
        bce = (sumA + (NPIX - nM) * LOG2 - sumB_Mt) / NPIX
        inter = sumS_Mt + 0.5 * (nf - n_Mt)
        sumP = sumS_M + 0.5 * (NPIX - nM)
        dice = 1.0 - (2.0 * inter + 1.0) / (sumP + nf + 1.0)
        extra = (sumB_t - sumB_Mt) / nf
        contrib = bce + dice + extra
        res = res + jnp.where(n > 0, contrib, 0.0)

    n_t_total = jnp.int32(0)
    for t in range(5):
        cnt_t = (n_ct[0][t] + n_ct[1][t] + n_ct[2][t] + n_ct[3][t]
                 + n_ct[4][t])
        n_t_total = n_t_total + (cnt_t > 0).astype(jnp.int32)

    o_ref[0, 0] = res / (n_t_total * 2 + 1).astype(jnp.float32)


@functools.partial(jax.jit, static_argnames=("interpret",))
def _run(pred_out, target_mask, interpret=False):
    p = pred_out.reshape(5, H, W)
    tgt = target_mask.reshape(H, W)
    out = pl.pallas_call(
        _loss_kernel,
        out_shape=jax.ShapeDtypeStruct((1, 1), jnp.float32),
        in_specs=[
            pl.BlockSpec(memory_space=pltpu.VMEM),
            pl.BlockSpec(memory_space=pltpu.VMEM),
        ],
        out_specs=pl.BlockSpec(memory_space=pltpu.SMEM),
        interpret=interpret,
    )(p, tgt)
    return out[0, 0]


def kernel(pred_out, target_mask):
    return _run(pred_out, target_mask)
